# baseline (speedup 1.0000x reference)
import functools

import jax
import jax.numpy as jnp
from jax import lax
from jax.experimental import pallas as pl
from jax.experimental.pallas import tpu as pltpu
from jax.experimental.pallas import tpu_sc as plsc

B, L, H, V = 1024, 200, 64, 1000002

NC, NS = 2, 16
NW = NC * NS
UB = 128
NCH = (B * L) // UB
UPW = NCH // NW
RING = 5
LOOK = 3

_sc_mesh = plsc.VectorSubcoreMesh(core_axis_name="c", subcore_axis_name="s")


@functools.partial(
    pl.kernel,
    mesh=_sc_mesh,
    out_type=jax.ShapeDtypeStruct((NCH, UB, H), jnp.float32),
    scratch_types=[
        pltpu.VMEM((UPW, UB), jnp.int32),
        pltpu.VMEM((RING, UB, H), jnp.float32),
        [pltpu.SemaphoreType.DMA] * RING,
        [pltpu.SemaphoreType.DMA] * RING,
    ],
    compiler_params=pltpu.CompilerParams(use_tc_tiling_on_sc=False),
)
def _gather_sc(x_hbm, tok_hbm, out_hbm, idx_v, rows, gsem, ssem):
    wid = lax.axis_index("s") * NC + lax.axis_index("c")
    r0 = wid * UPW
    pltpu.sync_copy(x_hbm.at[pl.ds(r0, UPW)], idx_v)

    def prep(i, s, may_wait):
        if may_wait:
            @pl.when(i >= RING)
            def _():
                pltpu.make_async_copy(rows.at[s], out_hbm.at[0],
                                      ssem[s]).wait()
        pltpu.async_copy(tok_hbm.at[idx_v.at[i]], rows.at[s], gsem[s])

    def consume(i, s):
        pltpu.make_async_copy(tok_hbm.at[idx_v.at[i]], rows.at[s],
                              gsem[s]).wait()
        pltpu.async_copy(rows.at[s], out_hbm.at[r0 + i], ssem[s])

    for s in range(LOOK):
        prep(s, s, False)

    def round_(jj, carry):
        for s in range(RING):
            i = jj * RING + s
            consume(i, s)
            jn = i + LOOK

            @pl.when(jn < UPW)
            def _():
                prep(jn, (s + LOOK) % RING, True)
        return carry

    lax.fori_loop(0, UPW // RING, round_, 0)
    for s in range(RING):
        pltpu.make_async_copy(rows.at[s], out_hbm.at[0], ssem[s]).wait()


_MB = 8


def _add_mask_body(tok_ref, pos_ref, x_ref, out_ref, m_ref):
    out_ref[...] = tok_ref[...] + pos_ref[...][None, :, :]
    xb = x_ref[...]
    m_ref[...] = jnp.broadcast_to((xb > 0)[:, None, None, :], (_MB, 1, L, L))


_add_mask_tc = pl.pallas_call(
    _add_mask_body,
    grid=(B // _MB,),
    in_specs=[
        pl.BlockSpec((_MB, L, H), lambda i: (i, 0, 0)),
        pl.BlockSpec((L, H), lambda i: (0, 0)),
        pl.BlockSpec((_MB, L), lambda i: (i, 0)),
    ],
    out_specs=[
        pl.BlockSpec((_MB, L, H), lambda i: (i, 0, 0)),
        pl.BlockSpec((_MB, 1, L, L), lambda i: (i, 0, 0, 0)),
    ],
    out_shape=(
        jax.ShapeDtypeStruct((B, L, H), jnp.float32),
        jax.ShapeDtypeStruct((B, 1, L, L), jnp.bool_),
    ),
)


def kernel(x, token_w, pos_w):
    x_flat = x.reshape(NCH, UB)
    tok = _gather_sc(x_flat, token_w).reshape(B, L, H)
    out, mask = _add_mask_tc(tok, pos_w, x)
    return out, mask


# speedup vs baseline: 1.5712x; 1.5712x over previous
import functools

import jax
import jax.numpy as jnp
from jax import lax
from jax.experimental import pallas as pl
from jax.experimental.pallas import tpu as pltpu
from jax.experimental.pallas import tpu_sc as plsc

B, L, H, V = 1024, 200, 64, 1000002
VH = V // 2

NC, NS = 2, 16
NW = NC * NS
UB = 128
BPL = B // UB
NUNITS = L * BPL
UPW = NUNITS // NW

_sc_mesh = plsc.VectorSubcoreMesh(core_axis_name="c", subcore_axis_name="s")


@functools.partial(
    pl.kernel,
    mesh=_sc_mesh,
    out_type=jax.ShapeDtypeStruct((L, H, B), jnp.float32),
    scratch_types=[
        pltpu.VMEM((2, 8, B), jnp.int32),
        pltpu.VMEM((2, UB), jnp.int32),
        pltpu.VMEM((2, UB), jnp.int32),
        pltpu.VMEM((2, UB, 128), jnp.float32),
        pltpu.VMEM((2, H, UB), jnp.float32),
        pltpu.VMEM((L, H), jnp.float32),
        [pltpu.SemaphoreType.DMA] * 2,
        [pltpu.SemaphoreType.DMA] * 2,
    ],
    compiler_params=pltpu.CompilerParams(needs_layout_passes=False),
)
def _embed_sc(x_hbm, tw_hbm, pos_hbm, out_hbm,
              idxg, idxs, colb, rows, outb, pos_v, gsem, ssem):
    wid = lax.axis_index("s") * NC + lax.axis_index("c")
    u0 = wid * UPW
    pltpu.sync_copy(pos_hbm, pos_v)
    ga = (u0 // BPL) // 8
    gb = ((u0 + UPW - 1) // BPL) // 8
    pltpu.sync_copy(x_hbm.at[ga], idxg.at[0])
    pltpu.sync_copy(x_hbm.at[gb], idxg.at[1])

    def prep(i, s):
        u = u0 + i
        l = u // BPL
        b0 = (u % BPL) * UB
        gi = (l // 8) - ga
        lr = l % 8
        for k in range(UB // 16):
            sl = pl.ds(k * 16, 16)
            vraw = idxg[gi, lr, pl.ds(b0 + k * 16, 16)]
            idxs[s, sl] = lax.shift_right_logical(vraw, 1)
            colb[s, sl] = lax.shift_left(vraw & 1, 6)
        pltpu.async_copy(tw_hbm.at[idxs.at[s]], rows.at[s], gsem[s])

    def consume(i, s):
        u = u0 + i
        l = u // BPL
        b0 = (u % BPL) * UB
        pltpu.make_async_copy(tw_hbm.at[idxs.at[s]], rows.at[s],
                              gsem[s]).wait()

        @pl.when(i >= 2)
        def _():
            pltpu.make_async_copy(outb.at[s], out_hbm.at[0, :, pl.ds(0, UB)],
                                  ssem[s]).wait()

        lsp = jnp.full((16,), l, jnp.int32)

        @plsc.parallel_loop(0, H)
        def hloop(h):
            hsp = jnp.full((16,), h, jnp.int32)
            psp = plsc.load_gather(pos_v, [lsp, hsp])
            for bb in range(UB // 16):
                rid = lax.iota(jnp.int32, 16) + (bb * 16)
                cid = colb[s, pl.ds(bb * 16, 16)] + h
                vals = plsc.load_gather(rows.at[s], [rid, cid])
                outb[s, h, pl.ds(bb * 16, 16)] = vals + psp

        pltpu.async_copy(outb.at[s], out_hbm.at[l, :, pl.ds(b0, UB)], ssem[s])

    prep(0, 0)

    def pair(ii, carry):
        i = ii * 2
        prep(i + 1, 1)
        consume(i, 0)

        @pl.when(i + 2 < UPW)
        def _():
            prep(i + 2, 0)

        consume(i + 1, 1)
        return carry

    lax.fori_loop(0, UPW // 2, pair, 0)
    for s in range(2):
        pltpu.make_async_copy(outb.at[s], out_hbm.at[0, :, pl.ds(0, UB)],
                              ssem[s]).wait()


def _mask_body(xT_ref, m_ref):
    m_ref[0, 0] = xT_ref[...] > 0


_mask_tc = pl.pallas_call(
    _mask_body,
    grid=(L,),
    in_specs=[pl.BlockSpec((L, B), lambda i: (0, 0))],
    out_specs=pl.BlockSpec((1, 1, L, B), lambda i: (0, i, 0, 0)),
    out_shape=jax.ShapeDtypeStruct((1, L, L, B), jnp.bool_),
)


def kernel(x, token_w, pos_w):
    xT = x.T
    xT3 = xT.reshape(L // 8, 8, B)
    tw128 = token_w.reshape(VH, 128)
    out_t = _embed_sc(xT3, tw128, pos_w)
    maskT = _mask_tc(xT)
    out = jnp.transpose(out_t, (2, 0, 1))
    mask = jnp.transpose(maskT, (3, 0, 1, 2))
    return out, mask
